# baseline (speedup 1.0000x reference)
import functools

import jax
import jax.numpy as jnp
from jax import lax
from jax.experimental import pallas as pl
from jax.experimental.pallas import tpu as pltpu
from jax.experimental.pallas import tpu_sc as plsc

D_IN = 2048
D_SAE = 65536
B_TOK = 64
K_TOP = 64

BN_ENC = 1024
R_SLAB = 8
NGRP = D_SAE // 16
_TINY = 1.1754944e-38
_TINY_BITS = 0x00800000


def _encode_body(acts_ref, w_ref, benc_ref, bdec_ref, out_ref):
    x = acts_ref[...] - bdec_ref[...]
    pre = jnp.dot(x, w_ref[...], preferred_element_type=jnp.float32)
    out_ref[...] = jnp.maximum(pre + benc_ref[...], 0.0)


def _thresh_body(a_ref, thr_ref, cnt_ref):
    ai = lax.bitcast_convert_type(a_ref[...], jnp.int32)
    ones = jnp.ones((D_SAE, 1), jnp.float32)

    def step(it, lo):
        j = 30 - it
        mid = lo + jnp.left_shift(jnp.int32(1), j)
        ind = (ai >= mid).astype(jnp.float32)
        cnt = jnp.dot(ind, ones, preferred_element_type=jnp.float32)
        return jnp.where(cnt >= float(K_TOP), mid, lo)

    lo = lax.fori_loop(0, 31, step, jnp.zeros((R_SLAB, 1), jnp.int32))
    thr_ref[...] = jnp.broadcast_to(lax.bitcast_convert_type(lo, jnp.float32),
                                    (R_SLAB, 128))
    teff = jnp.maximum(lo, jnp.int32(_TINY_BITS))
    m = (ai >= teff).astype(jnp.float32)
    mm = m.reshape(R_SLAB * (D_SAE // 128), 128)
    li = lax.broadcasted_iota(jnp.int32, (128, 8), 0)
    gi = lax.broadcasted_iota(jnp.int32, (128, 8), 1)
    sel = (li // 16 == gi).astype(jnp.float32)
    cnt = jnp.dot(mm, sel, preferred_element_type=jnp.float32)
    cnt_ref[...] = cnt.astype(jnp.int32)


_SC_INFO = plsc.get_sparse_core_info()
_NC = _SC_INFO.num_cores
_NS = _SC_INFO.num_subcores
_NW = _NC * _NS
_ROWS_PER_W = B_TOK // _NW
_CAP = 112
_GB = 8

_GD = lax.GatherDimensionNumbers(offset_dims=(), collapsed_slice_dims=(0,),
                                 start_index_map=(0,))


def _splat_lane(grp, k):
    idx = jnp.full((16, 1), 0, jnp.int32) + k
    return lax.gather(grp, idx, _GD, slice_sizes=(1,),
                      mode=lax.GatherScatterMode.PROMISE_IN_BOUNDS)


def _sc_decode_body(a_hbm, thr_hbm, cnt_hbm, wdec_hbm, bdec_hbm, out_hbm,
                    arow, cntv, idxbuf, valbuf, wrows, recon, thrv,
                    sem, asem, gsem, gsem2):
    wid = lax.axis_index("s") * _NC + lax.axis_index("c")
    iota16 = lax.iota(jnp.int32, 16)
    pltpu.sync_copy(thr_hbm, thrv)

    def shuf_reduce(x, op):
        for sh in (1, 2, 4, 8):
            perm = jnp.bitwise_xor(iota16, sh).reshape(16, 1)
            x = op(x, lax.gather(x, perm, _GD, slice_sizes=(1,),
                                 mode=lax.GatherScatterMode.PROMISE_IN_BOUNDS))
        return x

    def do_row(r, _):
        row = wid * _ROWS_PER_W + r
        tg = thrv[pl.ds(row * 16, 16)]
        ts = jnp.maximum(tg[0], _TINY)
        tsv = jnp.full((16,), ts, jnp.float32)
        hbdec = pltpu.async_copy(bdec_hbm, recon, sem)
        nsl = 4
        slc = D_SAE // nsl
        hsl = [pltpu.async_copy(a_hbm.at[pl.ds(row * D_SAE + i * slc, slc)],
                                arow.at[pl.ds(i * slc, slc)], asem)
               for i in range(nsl)]
        pltpu.sync_copy(cnt_hbm.at[pl.ds(row * NGRP, NGRP)], cntv)

        def cb_body(cb, c):
            cnts = cntv[pl.ds(cb * 16, 16)]
            mx = shuf_reduce(cnts, jnp.maximum)

            def hit_cb(c):
                for gj in range(16):
                    g = cb * 16 + gj

                    def hit(c, g=g, cg=cnts[gj]):
                        v = arow[pl.ds(g * 16, 16)]
                        m = v >= tsv
                        gb = jnp.full((16,), g * 16, jnp.int32)

                        def take_one(j, carry, v=v, gb=gb):
                            c, m = carry
                            lsp = shuf_reduce(jnp.where(m, iota16,
                                                        jnp.full((16,), 16,
                                                                 jnp.int32)),
                                              jnp.minimum)
                            l0 = lsp[0]
                            vs = lax.gather(
                                v, jnp.full((16, 1), 0, jnp.int32) + l0, _GD,
                                slice_sizes=(1,),
                                mode=lax.GatherScatterMode.PROMISE_IN_BOUNDS)
                            cc = jnp.minimum(c, _CAP)
                            valbuf[pl.ds(cc, 16)] = vs
                            idxbuf[pl.ds(cc, 16)] = gb + lsp
                            return (c + 1, m & (iota16 != lsp))

                        c, _m = lax.fori_loop(0, jnp.minimum(cg, 16), take_one,
                                              (c, m))
                        return c

                    c = lax.cond(cnts[gj] > 0, hit, lambda c: c, c)
                return c

            return lax.cond(mx[0] > 0, hit_cb, lambda c: c, c)

        c = jnp.int32(0)
        cb_per_sl = (NGRP // 16) // nsl
        for i in range(nsl):
            hsl[i].wait()
            c = lax.fori_loop(i * cb_per_sl, (i + 1) * cb_per_sl, cb_body, c)
        hbdec.wait()

        cs = jnp.full((16,), c, jnp.int32)
        for t in range(K_TOP // 16):
            live = (jnp.full((16,), 16 * t, jnp.int32) + iota16) < cs
            idxbuf[pl.ds(16 * t, 16)] = jnp.where(
                live, idxbuf[pl.ds(16 * t, 16)], jnp.zeros((16,), jnp.int32))
            valbuf[pl.ds(16 * t, 16)] = jnp.where(
                live, valbuf[pl.ds(16 * t, 16)], jnp.zeros((16,), jnp.float32))

        nbatch = K_TOP // _GB
        gsems = (gsem, gsem2)
        handles = [None, None]
        handles[0] = pltpu.async_copy(
            wdec_hbm.at[idxbuf.at[pl.ds(0, _GB)]], wrows.at[0], gsems[0])
        for g in range(nbatch):
            buf = g % 2
            if g + 1 < nbatch:
                handles[1 - buf] = pltpu.async_copy(
                    wdec_hbm.at[idxbuf.at[pl.ds((g + 1) * _GB, _GB)]],
                    wrows.at[1 - buf], gsems[1 - buf])
            handles[buf].wait()
            vgrp = valbuf[pl.ds(g * _GB, 16)]
            vks = [_splat_lane(vgrp, jnp.int32(k)) for k in range(_GB)]

            def fma_blk(b, _, vks=vks, buf=buf):
                for u in range(8):
                    off = b * 128 + u * 16
                    acc = recon[pl.ds(off, 16)]
                    for k in range(_GB):
                        acc = acc + vks[k] * wrows[buf, k, pl.ds(off, 16)]
                    recon[pl.ds(off, 16)] = acc
                return 0

            lax.fori_loop(0, D_IN // 128, fma_blk, 0)

        pltpu.sync_copy(recon, out_hbm.at[pl.ds(row * D_IN, D_IN)])
        return 0

    lax.fori_loop(0, _ROWS_PER_W, do_row, 0)


def _sc_decode(a_flat, thr_flat, cnt_flat, W_dec, b_dec):
    mesh = plsc.VectorSubcoreMesh(core_axis_name="c", subcore_axis_name="s")
    f = functools.partial(
        pl.kernel,
        mesh=mesh,
        out_type=jax.ShapeDtypeStruct((B_TOK * D_IN,), jnp.float32),
        scratch_types=[
            pltpu.VMEM((D_SAE,), jnp.float32),
            pltpu.VMEM((NGRP,), jnp.int32),
            pltpu.VMEM((128,), jnp.int32),
            pltpu.VMEM((128,), jnp.float32),
            pltpu.VMEM((2, _GB, D_IN), jnp.float32),
            pltpu.VMEM((D_IN,), jnp.float32),
            pltpu.VMEM((B_TOK * 16,), jnp.float32),
            pltpu.SemaphoreType.DMA,
            pltpu.SemaphoreType.DMA,
            pltpu.SemaphoreType.DMA,
            pltpu.SemaphoreType.DMA,
        ],
    )(_sc_decode_body)
    return f(a_flat, thr_flat, cnt_flat, W_dec, b_dec)


def kernel(acts, W_enc, W_dec, b_enc, b_dec):
    b_enc2 = b_enc.reshape(1, D_SAE)
    b_dec2 = b_dec.reshape(1, D_IN)

    a = pl.pallas_call(
        _encode_body,
        grid=(D_SAE // BN_ENC,),
        in_specs=[
            pl.BlockSpec((B_TOK, D_IN), lambda i: (0, 0)),
            pl.BlockSpec((D_IN, BN_ENC), lambda i: (0, i)),
            pl.BlockSpec((1, BN_ENC), lambda i: (0, i)),
            pl.BlockSpec((1, D_IN), lambda i: (0, 0)),
        ],
        out_specs=pl.BlockSpec((B_TOK, BN_ENC), lambda i: (0, i)),
        out_shape=jax.ShapeDtypeStruct((B_TOK, D_SAE), jnp.float32),
    )(acts, W_enc, b_enc2, b_dec2)

    thresh, cnt16 = pl.pallas_call(
        _thresh_body,
        grid=(B_TOK // R_SLAB,),
        in_specs=[pl.BlockSpec((R_SLAB, D_SAE), lambda i: (i, 0))],
        out_specs=[pl.BlockSpec((R_SLAB, 128), lambda i: (i, 0)),
                   pl.BlockSpec((R_SLAB * (D_SAE // 128), 8), lambda i: (i, 0))],
        out_shape=[jax.ShapeDtypeStruct((B_TOK, 128), jnp.float32),
                   jax.ShapeDtypeStruct((B_TOK * (D_SAE // 128), 8), jnp.int32)],
    )(a)

    recon_flat = _sc_decode(a.reshape(-1), thresh[:, :16].reshape(-1),
                            cnt16.reshape(-1), W_dec, b_dec)
    return recon_flat.reshape(B_TOK, D_IN)


# speedup vs baseline: 1.9014x; 1.9014x over previous
import functools

import jax
import jax.numpy as jnp
from jax import lax
from jax.experimental import pallas as pl
from jax.experimental.pallas import tpu as pltpu
from jax.experimental.pallas import tpu_sc as plsc

D_IN = 2048
D_SAE = 65536
B_TOK = 64
K_TOP = 64

BN_ENC = 1024
R_SLAB = 8
NGRP = D_SAE // 16
_TINY = 1.1754944e-38
_TINY_BITS = 0x00800000


def _encode_body(acts_ref, w_ref, benc_ref, bdec_ref, out_ref):
    x = acts_ref[...] - bdec_ref[...]
    pre = jnp.dot(x, w_ref[...], preferred_element_type=jnp.float32)
    out_ref[...] = jnp.maximum(pre + benc_ref[...], 0.0)


def _thresh_body(a_ref, thr_ref, cnt_ref):
    ai = lax.bitcast_convert_type(a_ref[...], jnp.int32)

    def step(it, lo):
        j = 30 - it
        mid = lo + jnp.left_shift(jnp.int32(1), j)
        cnt = jnp.sum((ai >= mid).astype(jnp.int32), axis=1, keepdims=True)
        return jnp.where(cnt >= K_TOP, mid, lo)

    lo = lax.fori_loop(0, 31, step, jnp.zeros((R_SLAB, 1), jnp.int32))
    thr_ref[...] = jnp.broadcast_to(lax.bitcast_convert_type(lo, jnp.float32),
                                    (R_SLAB, 128))
    teff = jnp.maximum(lo, jnp.int32(_TINY_BITS))
    m = (ai >= teff).astype(jnp.float32)
    mm = m.reshape(R_SLAB * (D_SAE // 128), 128)
    li = lax.broadcasted_iota(jnp.int32, (128, 8), 0)
    gi = lax.broadcasted_iota(jnp.int32, (128, 8), 1)
    sel = (li // 16 == gi).astype(jnp.float32)
    cnt = jnp.dot(mm, sel, preferred_element_type=jnp.float32)
    cnt_ref[...] = cnt.astype(jnp.int32)


_SC_INFO = plsc.get_sparse_core_info()
_NC = _SC_INFO.num_cores
_NS = _SC_INFO.num_subcores
_NW = _NC * _NS
_ROWS_PER_W = B_TOK // _NW
_CAP = 112
_GB = 8

_GD = lax.GatherDimensionNumbers(offset_dims=(), collapsed_slice_dims=(0,),
                                 start_index_map=(0,))


def _splat_lane(grp, k):
    idx = jnp.full((16, 1), 0, jnp.int32) + k
    return lax.gather(grp, idx, _GD, slice_sizes=(1,),
                      mode=lax.GatherScatterMode.PROMISE_IN_BOUNDS)


def _sc_decode_body(a_hbm, thr_hbm, cnt_hbm, wdec_hbm, bdec_hbm, out_hbm,
                    arow, cntv, idxbuf, valbuf, wrows, recon, thrv,
                    sem, asem, gsem, gsem2):
    wid = lax.axis_index("s") * _NC + lax.axis_index("c")
    iota16 = lax.iota(jnp.int32, 16)
    pltpu.sync_copy(thr_hbm, thrv)

    def shuf_reduce(x, op):
        for sh in (1, 2, 4, 8):
            perm = jnp.bitwise_xor(iota16, sh).reshape(16, 1)
            x = op(x, lax.gather(x, perm, _GD, slice_sizes=(1,),
                                 mode=lax.GatherScatterMode.PROMISE_IN_BOUNDS))
        return x

    def do_row(r, _):
        row = wid * _ROWS_PER_W + r
        tg = thrv[pl.ds(row * 16, 16)]
        ts = jnp.maximum(tg[0], _TINY)
        tsv = jnp.full((16,), ts, jnp.float32)
        hbdec = pltpu.async_copy(bdec_hbm, recon, sem)
        nsl = 4
        slc = D_SAE // nsl
        hsl = [pltpu.async_copy(a_hbm.at[pl.ds(row * D_SAE + i * slc, slc)],
                                arow.at[pl.ds(i * slc, slc)], asem)
               for i in range(nsl)]
        pltpu.sync_copy(cnt_hbm.at[pl.ds(row * NGRP, NGRP)], cntv)

        def cb_body(cb, c):
            cnts = cntv[pl.ds(cb * 16, 16)]
            mx = shuf_reduce(cnts, jnp.maximum)

            def hit_cb(c):
                for gj in range(16):
                    g = cb * 16 + gj

                    def hit(c, g=g, cg=cnts[gj]):
                        v = arow[pl.ds(g * 16, 16)]
                        m = v >= tsv
                        gb = jnp.full((16,), g * 16, jnp.int32)

                        def take_one(j, carry, v=v, gb=gb):
                            c, m = carry
                            lsp = shuf_reduce(jnp.where(m, iota16,
                                                        jnp.full((16,), 16,
                                                                 jnp.int32)),
                                              jnp.minimum)
                            l0 = lsp[0]
                            vs = lax.gather(
                                v, jnp.full((16, 1), 0, jnp.int32) + l0, _GD,
                                slice_sizes=(1,),
                                mode=lax.GatherScatterMode.PROMISE_IN_BOUNDS)
                            cc = jnp.minimum(c, _CAP)
                            valbuf[pl.ds(cc, 16)] = vs
                            idxbuf[pl.ds(cc, 16)] = gb + lsp
                            return (c + 1, m & (iota16 != lsp))

                        c, _m = lax.fori_loop(0, jnp.minimum(cg, 16), take_one,
                                              (c, m))
                        return c

                    c = lax.cond(cnts[gj] > 0, hit, lambda c: c, c)
                return c

            return lax.cond(mx[0] > 0, hit_cb, lambda c: c, c)

        c = jnp.int32(0)
        cb_per_sl = (NGRP // 16) // nsl
        for i in range(nsl):
            hsl[i].wait()
            c = lax.fori_loop(i * cb_per_sl, (i + 1) * cb_per_sl, cb_body, c)
        hbdec.wait()

        cs = jnp.full((16,), c, jnp.int32)
        for t in range(K_TOP // 16):
            live = (jnp.full((16,), 16 * t, jnp.int32) + iota16) < cs
            idxbuf[pl.ds(16 * t, 16)] = jnp.where(
                live, idxbuf[pl.ds(16 * t, 16)], jnp.zeros((16,), jnp.int32))
            valbuf[pl.ds(16 * t, 16)] = jnp.where(
                live, valbuf[pl.ds(16 * t, 16)], jnp.zeros((16,), jnp.float32))

        nbatch = K_TOP // _GB
        gsems = (gsem, gsem2)
        handles = [None, None]
        handles[0] = pltpu.async_copy(
            wdec_hbm.at[idxbuf.at[pl.ds(0, _GB)]], wrows.at[0], gsems[0])
        for g in range(nbatch):
            buf = g % 2
            if g + 1 < nbatch:
                handles[1 - buf] = pltpu.async_copy(
                    wdec_hbm.at[idxbuf.at[pl.ds((g + 1) * _GB, _GB)]],
                    wrows.at[1 - buf], gsems[1 - buf])
            handles[buf].wait()
            vgrp = valbuf[pl.ds(g * _GB, 16)]
            vks = [_splat_lane(vgrp, jnp.int32(k)) for k in range(_GB)]

            def fma_blk(b, _, vks=vks, buf=buf):
                for u in range(8):
                    off = b * 128 + u * 16
                    acc = recon[pl.ds(off, 16)]
                    for k in range(_GB):
                        acc = acc + vks[k] * wrows[buf, k, pl.ds(off, 16)]
                    recon[pl.ds(off, 16)] = acc
                return 0

            lax.fori_loop(0, D_IN // 128, fma_blk, 0)

        pltpu.sync_copy(recon, out_hbm.at[pl.ds(row * D_IN, D_IN)])
        return 0

    lax.fori_loop(0, _ROWS_PER_W, do_row, 0)


def _sc_decode(a_flat, thr_flat, cnt_flat, W_dec, b_dec):
    mesh = plsc.VectorSubcoreMesh(core_axis_name="c", subcore_axis_name="s")
    f = functools.partial(
        pl.kernel,
        mesh=mesh,
        out_type=jax.ShapeDtypeStruct((B_TOK * D_IN,), jnp.float32),
        scratch_types=[
            pltpu.VMEM((D_SAE,), jnp.float32),
            pltpu.VMEM((NGRP,), jnp.int32),
            pltpu.VMEM((128,), jnp.int32),
            pltpu.VMEM((128,), jnp.float32),
            pltpu.VMEM((2, _GB, D_IN), jnp.float32),
            pltpu.VMEM((D_IN,), jnp.float32),
            pltpu.VMEM((B_TOK * 16,), jnp.float32),
            pltpu.SemaphoreType.DMA,
            pltpu.SemaphoreType.DMA,
            pltpu.SemaphoreType.DMA,
            pltpu.SemaphoreType.DMA,
        ],
    )(_sc_decode_body)
    return f(a_flat, thr_flat, cnt_flat, W_dec, b_dec)


def kernel(acts, W_enc, W_dec, b_enc, b_dec):
    b_enc2 = b_enc.reshape(1, D_SAE)
    b_dec2 = b_dec.reshape(1, D_IN)

    a = pl.pallas_call(
        _encode_body,
        grid=(D_SAE // BN_ENC,),
        in_specs=[
            pl.BlockSpec((B_TOK, D_IN), lambda i: (0, 0)),
            pl.BlockSpec((D_IN, BN_ENC), lambda i: (0, i)),
            pl.BlockSpec((1, BN_ENC), lambda i: (0, i)),
            pl.BlockSpec((1, D_IN), lambda i: (0, 0)),
        ],
        out_specs=pl.BlockSpec((B_TOK, BN_ENC), lambda i: (0, i)),
        out_shape=jax.ShapeDtypeStruct((B_TOK, D_SAE), jnp.float32),
    )(acts, W_enc, b_enc2, b_dec2)

    thresh, cnt16 = pl.pallas_call(
        _thresh_body,
        grid=(B_TOK // R_SLAB,),
        in_specs=[pl.BlockSpec((R_SLAB, D_SAE), lambda i: (i, 0))],
        out_specs=[pl.BlockSpec((R_SLAB, 128), lambda i: (i, 0)),
                   pl.BlockSpec((R_SLAB * (D_SAE // 128), 8), lambda i: (i, 0))],
        out_shape=[jax.ShapeDtypeStruct((B_TOK, 128), jnp.float32),
                   jax.ShapeDtypeStruct((B_TOK * (D_SAE // 128), 8), jnp.int32)],
    )(a)

    recon_flat = _sc_decode(a.reshape(-1), thresh[:, :16].reshape(-1),
                            cnt16.reshape(-1), W_dec, b_dec)
    return recon_flat.reshape(B_TOK, D_IN)
